# baseline (speedup 1.0000x reference)
import functools

import jax
import jax.numpy as jnp
from jax import lax
from jax.experimental import pallas as pl
from jax.experimental.pallas import tpu as pltpu
from jax.experimental.pallas import tpu_sc as plsc

NUM_CORES = 2
NUM_SUBCORES = 16
NUM_WORKERS = NUM_CORES * NUM_SUBCORES
LANES = 16
NB = 128


TQ = 384
TR = TQ // 2


@functools.partial(jax.jit, static_argnames=("v", "d"))
def _transpose_table(wT, tail, *, v, d):
    vmain = (v // TQ) * TQ if (v % TQ) else v - TQ * 0
    nchunks = vmain // TQ
    tail_rows = tail.shape[0]
    mesh = plsc.VectorSubcoreMesh(core_axis_name="c", subcore_axis_name="s")

    @functools.partial(
        pl.kernel,
        out_type=jax.ShapeDtypeStruct((v // 2, 2 * d), jnp.float32),
        mesh=mesh,
        scratch_types=[
            [pltpu.VMEM((d, TQ), jnp.float32) for _ in range(2)],
            [pltpu.VMEM((TR, 2 * d), jnp.float32) for _ in range(2)],
            pltpu.VMEM((tail_rows, 2 * d), jnp.float32),
            [pltpu.SemaphoreType.DMA for _ in range(2)],
            [pltpu.SemaphoreType.DMA for _ in range(2)],
        ],
        compiler_params=pltpu.CompilerParams(
            use_tc_tiling_on_sc=True, needs_layout_passes=False
        ),
    )
    def body(wT_hbm, tail_hbm, out_hbm, combs, obufs, tailv, isems, osems):
        cid = lax.axis_index("c")
        sid = lax.axis_index("s")
        wid = sid * NUM_CORES + cid

        @pl.when(wid == 0)
        def _():
            pltpu.sync_copy(tail_hbm, tailv)
            pltpu.sync_copy(tailv, out_hbm.at[pl.ds(vmain // 2, tail_rows)])

        iota = lax.iota(jnp.int32, LANES)
        cvecs = tuple(iota + c0 for c0 in range(0, d, LANES))
        nj = d // LANES
        steps = (nchunks + NUM_WORKERS - 1) // NUM_WORKERS
        last = nchunks - 1

        def ifire(t, b):
            i = jnp.minimum(t * NUM_WORKERS + wid, last)
            for k in range(d // 8):
                pltpu.async_copy(
                    wT_hbm.at[pl.ds(8 * k, 8), pl.ds(i * TQ, TQ)],
                    combs[b].at[pl.ds(8 * k, 8)],
                    isems[b],
                )

        def iwait(t, b):
            i = jnp.minimum(t * NUM_WORKERS + wid, last)
            for k in range(d // 8):
                pltpu.make_async_copy(
                    wT_hbm.at[pl.ds(8 * k, 8), pl.ds(i * TQ, TQ)],
                    combs[b].at[pl.ds(8 * k, 8)],
                    isems[b],
                ).wait()

        def ofire(t, b):
            i = jnp.minimum(t * NUM_WORKERS + wid, last)
            pltpu.async_copy(obufs[b], out_hbm.at[pl.ds(i * TR, TR)], osems[b])

        def owait(t, b):
            i = jnp.minimum(t * NUM_WORKERS + wid, last)
            pltpu.make_async_copy(
                obufs[b], out_hbm.at[pl.ds(i * TR, TR)], osems[b]
            ).wait()

        def transpose(b):
            @plsc.parallel_loop(0, TR, unroll=4)
            def row(r):
                q0 = jnp.zeros((LANES,), jnp.int32) + 2 * r
                for half in range(2):
                    for j in range(nj):
                        vals = plsc.load_gather(combs[b], [cvecs[j], q0 + half])
                        obufs[b][r, pl.ds((half * nj + j) * LANES, LANES)] = vals

        ifire(0, 0)

        def step(t, b):
            @pl.when(t + 1 < steps)
            def _():
                ifire(t + 1, 1 - b)

            iwait(t, b)

            @pl.when(t >= 2)
            def _():
                owait(t - 2, b)

            transpose(b)
            ofire(t, b)

        def group(g, carry):
            for b in range(2):
                step(2 * g + b, b)
            return carry

        lax.fori_loop(0, steps // 2, group, 0)
        for t in range(steps - steps % 2, steps):
            step(t, t % 2)

        owait(steps - 2, (steps - 2) % 2)
        owait(steps - 1, (steps - 1) % 2)

    return body(wT, tail)


@functools.partial(jax.jit, static_argnames=("hist", "d"))
def _gather_rows(vidx, csel, wv, *, hist, d):
    batch = vidx.shape[1]
    mesh = plsc.VectorSubcoreMesh(core_axis_name="c", subcore_axis_name="s")

    @functools.partial(
        pl.kernel,
        out_type=jax.ShapeDtypeStruct((hist, d, batch), jnp.float32),
        mesh=mesh,
        scratch_types=[
            pltpu.VMEM((hist, NB), jnp.int32),
            pltpu.VMEM((hist, NB), jnp.int32),
            [pltpu.VMEM((NB, 2 * d), jnp.float32) for _ in range(4)],
            [pltpu.VMEM((d, NB), jnp.float32) for _ in range(2)],
            [pltpu.SemaphoreType.DMA for _ in range(4)],
            [pltpu.SemaphoreType.DMA for _ in range(2)],
        ],
        compiler_params=pltpu.CompilerParams(
            use_tc_tiling_on_sc=True, needs_layout_passes=False
        ),
    )
    def body(vidx_hbm, csel_hbm, wv_hbm, out_hbm, iv, cv, gbufs, obufs, gsems, ssems):
        cid = lax.axis_index("c")
        sid = lax.axis_index("s")
        wid = sid * NUM_CORES + cid
        n0 = wid * NB
        pltpu.sync_copy(vidx_hbm.at[:, pl.ds(n0, NB)], iv)
        pltpu.sync_copy(csel_hbm.at[:, pl.ds(n0, NB)], cv)

        iota = lax.iota(jnp.int32, LANES)

        def gfire(h, b):
            pltpu.async_copy(wv_hbm.at[iv.at[h]], gbufs[b], gsems[b])

        def gwait(h, b):
            pltpu.make_async_copy(wv_hbm.at[iv.at[h]], gbufs[b], gsems[b]).wait()

        def sfire(h, b):
            pltpu.async_copy(obufs[b], out_hbm.at[h, :, pl.ds(n0, NB)], ssems[b])

        def swait(h, b):
            pltpu.make_async_copy(
                obufs[b], out_hbm.at[h, :, pl.ds(n0, NB)], ssems[b]
            ).wait()

        nlvecs = tuple(iota + nl0 for nl0 in range(0, NB, LANES))

        def compact(h, b, ob):
            csels = tuple(cv[h, pl.ds(nl0, LANES)] for nl0 in range(0, NB, LANES))

            @plsc.parallel_loop(0, d, unroll=8)
            def col(c):
                for g in range(NB // LANES):
                    vals = plsc.load_gather(gbufs[b], [nlvecs[g], csels[g] + c])
                    obufs[ob][c, pl.ds(g * LANES, LANES)] = vals

        for b in range(4):
            gfire(b, b)

        def step(h, b, ob):
            @pl.when(h >= 2)
            def _():
                swait(h - 2, ob)

            gwait(h, b)
            compact(h, b, ob)
            sfire(h, ob)

            @pl.when(h + 4 < hist)
            def _():
                gfire(h + 4, b)

        def group(g, carry):
            for b in range(4):
                h = 4 * g + b
                step(h, b, b % 2)
            return carry

        lax.fori_loop(0, hist // 4, group, 0)
        for h in range(hist - hist % 4, hist):
            step(h, h % 4, h % 2)

        swait(hist - 2, (hist - 2) % 2)
        swait(hist - 1, (hist - 1) % 2)

    return body(vidx, csel, wv)


def kernel(input, weight):
    batch, hist = input.shape
    v, d = weight.shape
    assert batch == NUM_WORKERS * NB and hist % 2 == 0
    idxT = input.T
    vidx = idxT >> 1
    csel = (idxT & 1) * d
    vmain = (v // TQ) * TQ
    tail = weight[vmain:].reshape(-1, 2 * d)
    wv = _transpose_table(weight.T, tail, v=v, d=d)
    out3 = _gather_rows(vidx, csel, wv, hist=hist, d=d)
    return jnp.transpose(out3, (2, 0, 1))


# speedup vs baseline: 1.3591x; 1.3591x over previous
import functools

import jax
import jax.numpy as jnp
from jax import lax
from jax.experimental import pallas as pl
from jax.experimental.pallas import tpu as pltpu
from jax.experimental.pallas import tpu_sc as plsc

NUM_CORES = 2
NUM_SUBCORES = 16
NUM_WORKERS = NUM_CORES * NUM_SUBCORES
LANES = 16
NB = 128


TCQ = 2048


@functools.partial(jax.jit, static_argnames=("v", "d"))
def _tc_transpose(wT, *, v, d):
    def body(x_ref, o_ref):
        x = x_ref[...]
        y = x.T.reshape(TCQ // 2, 2, d)
        o_ref[:, 0:d] = y[:, 0, :]
        o_ref[:, d : 2 * d] = y[:, 1, :]

    return pl.pallas_call(
        body,
        grid=(pl.cdiv(v, TCQ),),
        in_specs=[pl.BlockSpec((d, TCQ), lambda i: (0, i))],
        out_specs=pl.BlockSpec((TCQ // 2, 2 * d), lambda i: (i, 0)),
        out_shape=jax.ShapeDtypeStruct((v // 2, 2 * d), jnp.float32),
    )(wT)


TQ = 384
TR = TQ // 2


@functools.partial(jax.jit, static_argnames=("v", "d"))
def _transpose_table(wT, tail, *, v, d):
    vmain = (v // TQ) * TQ if (v % TQ) else v - TQ * 0
    nchunks = vmain // TQ
    tail_rows = tail.shape[0]
    mesh = plsc.VectorSubcoreMesh(core_axis_name="c", subcore_axis_name="s")

    @functools.partial(
        pl.kernel,
        out_type=jax.ShapeDtypeStruct((v // 2, 2 * d), jnp.float32),
        mesh=mesh,
        scratch_types=[
            [pltpu.VMEM((d, TQ), jnp.float32) for _ in range(2)],
            [pltpu.VMEM((TR, 2 * d), jnp.float32) for _ in range(2)],
            pltpu.VMEM((tail_rows, 2 * d), jnp.float32),
            [pltpu.SemaphoreType.DMA for _ in range(2)],
            [pltpu.SemaphoreType.DMA for _ in range(2)],
        ],
        compiler_params=pltpu.CompilerParams(
            use_tc_tiling_on_sc=True, needs_layout_passes=False
        ),
    )
    def body(wT_hbm, tail_hbm, out_hbm, combs, obufs, tailv, isems, osems):
        cid = lax.axis_index("c")
        sid = lax.axis_index("s")
        wid = sid * NUM_CORES + cid

        @pl.when(wid == 0)
        def _():
            pltpu.sync_copy(tail_hbm, tailv)
            pltpu.sync_copy(tailv, out_hbm.at[pl.ds(vmain // 2, tail_rows)])

        iota = lax.iota(jnp.int32, LANES)
        cvecs = tuple(iota + c0 for c0 in range(0, d, LANES))
        nj = d // LANES
        steps = (nchunks + NUM_WORKERS - 1) // NUM_WORKERS
        last = nchunks - 1

        def ifire(t, b):
            i = jnp.minimum(t * NUM_WORKERS + wid, last)
            for k in range(d // 8):
                pltpu.async_copy(
                    wT_hbm.at[pl.ds(8 * k, 8), pl.ds(i * TQ, TQ)],
                    combs[b].at[pl.ds(8 * k, 8)],
                    isems[b],
                )

        def iwait(t, b):
            i = jnp.minimum(t * NUM_WORKERS + wid, last)
            for k in range(d // 8):
                pltpu.make_async_copy(
                    wT_hbm.at[pl.ds(8 * k, 8), pl.ds(i * TQ, TQ)],
                    combs[b].at[pl.ds(8 * k, 8)],
                    isems[b],
                ).wait()

        def ofire(t, b):
            i = jnp.minimum(t * NUM_WORKERS + wid, last)
            pltpu.async_copy(obufs[b], out_hbm.at[pl.ds(i * TR, TR)], osems[b])

        def owait(t, b):
            i = jnp.minimum(t * NUM_WORKERS + wid, last)
            pltpu.make_async_copy(
                obufs[b], out_hbm.at[pl.ds(i * TR, TR)], osems[b]
            ).wait()

        def transpose(b):
            @plsc.parallel_loop(0, TR, unroll=4)
            def row(r):
                q0 = jnp.zeros((LANES,), jnp.int32) + 2 * r
                for half in range(2):
                    for j in range(nj):
                        vals = plsc.load_gather(combs[b], [cvecs[j], q0 + half])
                        obufs[b][r, pl.ds((half * nj + j) * LANES, LANES)] = vals

        ifire(0, 0)

        def step(t, b):
            @pl.when(t + 1 < steps)
            def _():
                ifire(t + 1, 1 - b)

            iwait(t, b)

            @pl.when(t >= 2)
            def _():
                owait(t - 2, b)

            transpose(b)
            ofire(t, b)

        def group(g, carry):
            for b in range(2):
                step(2 * g + b, b)
            return carry

        lax.fori_loop(0, steps // 2, group, 0)
        for t in range(steps - steps % 2, steps):
            step(t, t % 2)

        owait(steps - 2, (steps - 2) % 2)
        owait(steps - 1, (steps - 1) % 2)

    return body(wT, tail)


@functools.partial(jax.jit, static_argnames=("hist", "d"))
def _gather_rows(vidx, csel, wv, *, hist, d):
    batch = vidx.shape[1]
    mesh = plsc.VectorSubcoreMesh(core_axis_name="c", subcore_axis_name="s")

    @functools.partial(
        pl.kernel,
        out_type=jax.ShapeDtypeStruct((hist, d, batch), jnp.float32),
        mesh=mesh,
        scratch_types=[
            pltpu.VMEM((hist, NB), jnp.int32),
            pltpu.VMEM((hist, NB), jnp.int32),
            [pltpu.VMEM((NB, 2 * d), jnp.float32) for _ in range(4)],
            [pltpu.VMEM((d, NB), jnp.float32) for _ in range(2)],
            [pltpu.SemaphoreType.DMA for _ in range(4)],
            [pltpu.SemaphoreType.DMA for _ in range(2)],
        ],
        compiler_params=pltpu.CompilerParams(
            use_tc_tiling_on_sc=True, needs_layout_passes=False
        ),
    )
    def body(vidx_hbm, csel_hbm, wv_hbm, out_hbm, iv, cv, gbufs, obufs, gsems, ssems):
        cid = lax.axis_index("c")
        sid = lax.axis_index("s")
        wid = sid * NUM_CORES + cid
        n0 = wid * NB
        pltpu.sync_copy(vidx_hbm.at[:, pl.ds(n0, NB)], iv)
        pltpu.sync_copy(csel_hbm.at[:, pl.ds(n0, NB)], cv)

        iota = lax.iota(jnp.int32, LANES)

        def gfire(h, b):
            pltpu.async_copy(wv_hbm.at[iv.at[h]], gbufs[b], gsems[b])

        def gwait(h, b):
            pltpu.make_async_copy(wv_hbm.at[iv.at[h]], gbufs[b], gsems[b]).wait()

        def sfire(h, b):
            pltpu.async_copy(obufs[b], out_hbm.at[h, :, pl.ds(n0, NB)], ssems[b])

        def swait(h, b):
            pltpu.make_async_copy(
                obufs[b], out_hbm.at[h, :, pl.ds(n0, NB)], ssems[b]
            ).wait()

        nlvecs = tuple(iota + nl0 for nl0 in range(0, NB, LANES))

        def compact(h, b, ob):
            csels = tuple(cv[h, pl.ds(nl0, LANES)] for nl0 in range(0, NB, LANES))

            @plsc.parallel_loop(0, d, unroll=8)
            def col(c):
                for g in range(NB // LANES):
                    vals = plsc.load_gather(gbufs[b], [nlvecs[g], csels[g] + c])
                    obufs[ob][c, pl.ds(g * LANES, LANES)] = vals

        for b in range(4):
            gfire(b, b)

        def step(h, b, ob):
            @pl.when(h >= 2)
            def _():
                swait(h - 2, ob)

            gwait(h, b)
            compact(h, b, ob)
            sfire(h, ob)

            @pl.when(h + 4 < hist)
            def _():
                gfire(h + 4, b)

        def group(g, carry):
            for b in range(4):
                h = 4 * g + b
                step(h, b, b % 2)
            return carry

        lax.fori_loop(0, hist // 4, group, 0)
        for h in range(hist - hist % 4, hist):
            step(h, h % 4, h % 2)

        swait(hist - 2, (hist - 2) % 2)
        swait(hist - 1, (hist - 1) % 2)

    return body(vidx, csel, wv)


def kernel(input, weight):
    batch, hist = input.shape
    v, d = weight.shape
    assert batch == NUM_WORKERS * NB and hist % 2 == 0
    idxT = input.T
    vidx = idxT >> 1
    csel = (idxT & 1) * d
    wv = _tc_transpose(weight.T, v=v, d=d)
    out3 = _gather_rows(vidx, csel, wv, hist=hist, d=d)
    return jnp.transpose(out3, (2, 0, 1))


# speedup vs baseline: 1.7248x; 1.2691x over previous
import functools

import jax
import jax.numpy as jnp
from jax import lax
from jax.experimental import pallas as pl
from jax.experimental.pallas import tpu as pltpu
from jax.experimental.pallas import tpu_sc as plsc

NUM_CORES = 2
NUM_SUBCORES = 16
NUM_WORKERS = NUM_CORES * NUM_SUBCORES
LANES = 16
NB = 128


TCQ = 8192


@functools.partial(jax.jit, static_argnames=("v", "d"))
def _tc_transpose(wT, *, v, d):
    def body(x_ref, o_ref):
        x = x_ref[...]
        y = x.T.reshape(TCQ // 2, 2, d)
        o_ref[:, 0:d] = y[:, 0, :]
        o_ref[:, d : 2 * d] = y[:, 1, :]

    return pl.pallas_call(
        body,
        grid=(pl.cdiv(v, TCQ),),
        in_specs=[pl.BlockSpec((d, TCQ), lambda i: (0, i))],
        out_specs=pl.BlockSpec((TCQ // 2, 2 * d), lambda i: (i, 0)),
        out_shape=jax.ShapeDtypeStruct((v // 2, 2 * d), jnp.float32),
    )(wT)


TQ = 384
TR = TQ // 2


@functools.partial(jax.jit, static_argnames=("v", "d"))
def _transpose_table(wT, tail, *, v, d):
    vmain = (v // TQ) * TQ if (v % TQ) else v - TQ * 0
    nchunks = vmain // TQ
    tail_rows = tail.shape[0]
    mesh = plsc.VectorSubcoreMesh(core_axis_name="c", subcore_axis_name="s")

    @functools.partial(
        pl.kernel,
        out_type=jax.ShapeDtypeStruct((v // 2, 2 * d), jnp.float32),
        mesh=mesh,
        scratch_types=[
            [pltpu.VMEM((d, TQ), jnp.float32) for _ in range(2)],
            [pltpu.VMEM((TR, 2 * d), jnp.float32) for _ in range(2)],
            pltpu.VMEM((tail_rows, 2 * d), jnp.float32),
            [pltpu.SemaphoreType.DMA for _ in range(2)],
            [pltpu.SemaphoreType.DMA for _ in range(2)],
        ],
        compiler_params=pltpu.CompilerParams(
            use_tc_tiling_on_sc=True, needs_layout_passes=False
        ),
    )
    def body(wT_hbm, tail_hbm, out_hbm, combs, obufs, tailv, isems, osems):
        cid = lax.axis_index("c")
        sid = lax.axis_index("s")
        wid = sid * NUM_CORES + cid

        @pl.when(wid == 0)
        def _():
            pltpu.sync_copy(tail_hbm, tailv)
            pltpu.sync_copy(tailv, out_hbm.at[pl.ds(vmain // 2, tail_rows)])

        iota = lax.iota(jnp.int32, LANES)
        cvecs = tuple(iota + c0 for c0 in range(0, d, LANES))
        nj = d // LANES
        steps = (nchunks + NUM_WORKERS - 1) // NUM_WORKERS
        last = nchunks - 1

        def ifire(t, b):
            i = jnp.minimum(t * NUM_WORKERS + wid, last)
            for k in range(d // 8):
                pltpu.async_copy(
                    wT_hbm.at[pl.ds(8 * k, 8), pl.ds(i * TQ, TQ)],
                    combs[b].at[pl.ds(8 * k, 8)],
                    isems[b],
                )

        def iwait(t, b):
            i = jnp.minimum(t * NUM_WORKERS + wid, last)
            for k in range(d // 8):
                pltpu.make_async_copy(
                    wT_hbm.at[pl.ds(8 * k, 8), pl.ds(i * TQ, TQ)],
                    combs[b].at[pl.ds(8 * k, 8)],
                    isems[b],
                ).wait()

        def ofire(t, b):
            i = jnp.minimum(t * NUM_WORKERS + wid, last)
            pltpu.async_copy(obufs[b], out_hbm.at[pl.ds(i * TR, TR)], osems[b])

        def owait(t, b):
            i = jnp.minimum(t * NUM_WORKERS + wid, last)
            pltpu.make_async_copy(
                obufs[b], out_hbm.at[pl.ds(i * TR, TR)], osems[b]
            ).wait()

        def transpose(b):
            @plsc.parallel_loop(0, TR, unroll=4)
            def row(r):
                q0 = jnp.zeros((LANES,), jnp.int32) + 2 * r
                for half in range(2):
                    for j in range(nj):
                        vals = plsc.load_gather(combs[b], [cvecs[j], q0 + half])
                        obufs[b][r, pl.ds((half * nj + j) * LANES, LANES)] = vals

        ifire(0, 0)

        def step(t, b):
            @pl.when(t + 1 < steps)
            def _():
                ifire(t + 1, 1 - b)

            iwait(t, b)

            @pl.when(t >= 2)
            def _():
                owait(t - 2, b)

            transpose(b)
            ofire(t, b)

        def group(g, carry):
            for b in range(2):
                step(2 * g + b, b)
            return carry

        lax.fori_loop(0, steps // 2, group, 0)
        for t in range(steps - steps % 2, steps):
            step(t, t % 2)

        owait(steps - 2, (steps - 2) % 2)
        owait(steps - 1, (steps - 1) % 2)

    return body(wT, tail)


@functools.partial(jax.jit, static_argnames=("hist", "d"))
def _gather_rows(vidx, csel, wv, *, hist, d):
    batch = vidx.shape[1]
    mesh = plsc.VectorSubcoreMesh(core_axis_name="c", subcore_axis_name="s")

    @functools.partial(
        pl.kernel,
        out_type=jax.ShapeDtypeStruct((hist, d, batch), jnp.float32),
        mesh=mesh,
        scratch_types=[
            pltpu.VMEM((hist, NB), jnp.int32),
            pltpu.VMEM((hist, NB), jnp.int32),
            [pltpu.VMEM((NB, 2 * d), jnp.float32) for _ in range(4)],
            [pltpu.VMEM((d, NB), jnp.float32) for _ in range(2)],
            [pltpu.SemaphoreType.DMA for _ in range(4)],
            [pltpu.SemaphoreType.DMA for _ in range(2)],
        ],
        compiler_params=pltpu.CompilerParams(
            use_tc_tiling_on_sc=True, needs_layout_passes=False
        ),
    )
    def body(vidx_hbm, csel_hbm, wv_hbm, out_hbm, iv, cv, gbufs, obufs, gsems, ssems):
        cid = lax.axis_index("c")
        sid = lax.axis_index("s")
        wid = sid * NUM_CORES + cid
        n0 = wid * NB
        pltpu.sync_copy(vidx_hbm.at[:, pl.ds(n0, NB)], iv)
        pltpu.sync_copy(csel_hbm.at[:, pl.ds(n0, NB)], cv)

        iota = lax.iota(jnp.int32, LANES)

        def gfire(h, b):
            pltpu.async_copy(wv_hbm.at[iv.at[h]], gbufs[b], gsems[b])

        def gwait(h, b):
            pltpu.make_async_copy(wv_hbm.at[iv.at[h]], gbufs[b], gsems[b]).wait()

        def sfire(h, b):
            pltpu.async_copy(obufs[b], out_hbm.at[h, :, pl.ds(n0, NB)], ssems[b])

        def swait(h, b):
            pltpu.make_async_copy(
                obufs[b], out_hbm.at[h, :, pl.ds(n0, NB)], ssems[b]
            ).wait()

        nlvecs = tuple(iota + nl0 for nl0 in range(0, NB, LANES))

        def compact(h, b, ob):
            csels = tuple(cv[h, pl.ds(nl0, LANES)] for nl0 in range(0, NB, LANES))

            @plsc.parallel_loop(0, d, unroll=8)
            def col(c):
                for g in range(NB // LANES):
                    vals = plsc.load_gather(gbufs[b], [nlvecs[g], csels[g] + c])
                    obufs[ob][c, pl.ds(g * LANES, LANES)] = vals

        for b in range(4):
            gfire(b, b)

        def step(h, b, ob):
            @pl.when(h >= 2)
            def _():
                swait(h - 2, ob)

            gwait(h, b)
            compact(h, b, ob)
            sfire(h, ob)

            @pl.when(h + 4 < hist)
            def _():
                gfire(h + 4, b)

        def group(g, carry):
            for b in range(4):
                h = 4 * g + b
                step(h, b, b % 2)
            return carry

        lax.fori_loop(0, hist // 4, group, 0)
        for h in range(hist - hist % 4, hist):
            step(h, h % 4, h % 2)

        swait(hist - 2, (hist - 2) % 2)
        swait(hist - 1, (hist - 1) % 2)

    return body(vidx, csel, wv)


def kernel(input, weight):
    batch, hist = input.shape
    v, d = weight.shape
    assert batch == NUM_WORKERS * NB and hist % 2 == 0
    idxT = input.T
    vidx = idxT >> 1
    csel = (idxT & 1) * d
    wv = _tc_transpose(weight.T, v=v, d=d)
    out3 = _gather_rows(vidx, csel, wv, hist=hist, d=d)
    return jnp.transpose(out3, (2, 0, 1))


# speedup vs baseline: 1.7480x; 1.0135x over previous
import functools

import jax
import jax.numpy as jnp
from jax import lax
from jax.experimental import pallas as pl
from jax.experimental.pallas import tpu as pltpu
from jax.experimental.pallas import tpu_sc as plsc

NUM_CORES = 2
NUM_SUBCORES = 16
NUM_WORKERS = NUM_CORES * NUM_SUBCORES
LANES = 16
NB = 128


TCQ = 16384


@functools.partial(jax.jit, static_argnames=("v", "d"))
def _tc_transpose(wT, *, v, d):
    def body(x_ref, o_ref):
        x = x_ref[...]
        y = x.T.reshape(TCQ // 2, 2, d)
        o_ref[:, 0:d] = y[:, 0, :]
        o_ref[:, d : 2 * d] = y[:, 1, :]

    return pl.pallas_call(
        body,
        grid=(pl.cdiv(v, TCQ),),
        in_specs=[pl.BlockSpec((d, TCQ), lambda i: (0, i))],
        out_specs=pl.BlockSpec((TCQ // 2, 2 * d), lambda i: (i, 0)),
        out_shape=jax.ShapeDtypeStruct((v // 2, 2 * d), jnp.float32),
    )(wT)


@functools.partial(jax.jit, static_argnames=("hist", "d"))
def _gather_rows(vidx, csel, wv, *, hist, d):
    batch = vidx.shape[1]
    mesh = plsc.VectorSubcoreMesh(core_axis_name="c", subcore_axis_name="s")

    @functools.partial(
        pl.kernel,
        out_type=jax.ShapeDtypeStruct((hist, d, batch), jnp.float32),
        mesh=mesh,
        scratch_types=[
            pltpu.VMEM((hist, NB), jnp.int32),
            pltpu.VMEM((hist, NB), jnp.int32),
            [pltpu.VMEM((NB, 2 * d), jnp.float32) for _ in range(4)],
            [pltpu.VMEM((d, NB), jnp.float32) for _ in range(2)],
            [pltpu.SemaphoreType.DMA for _ in range(4)],
            [pltpu.SemaphoreType.DMA for _ in range(2)],
        ],
        compiler_params=pltpu.CompilerParams(
            use_tc_tiling_on_sc=True, needs_layout_passes=False
        ),
    )
    def body(vidx_hbm, csel_hbm, wv_hbm, out_hbm, iv, cv, gbufs, obufs, gsems, ssems):
        cid = lax.axis_index("c")
        sid = lax.axis_index("s")
        wid = sid * NUM_CORES + cid
        n0 = wid * NB
        pltpu.sync_copy(vidx_hbm.at[:, pl.ds(n0, NB)], iv)
        pltpu.sync_copy(csel_hbm.at[:, pl.ds(n0, NB)], cv)

        iota = lax.iota(jnp.int32, LANES)

        def gfire(h, b):
            pltpu.async_copy(wv_hbm.at[iv.at[h]], gbufs[b], gsems[b])

        def gwait(h, b):
            pltpu.make_async_copy(wv_hbm.at[iv.at[h]], gbufs[b], gsems[b]).wait()

        def sfire(h, b):
            pltpu.async_copy(obufs[b], out_hbm.at[h, :, pl.ds(n0, NB)], ssems[b])

        def swait(h, b):
            pltpu.make_async_copy(
                obufs[b], out_hbm.at[h, :, pl.ds(n0, NB)], ssems[b]
            ).wait()

        nlvecs = tuple(iota + nl0 for nl0 in range(0, NB, LANES))

        def compact(h, b, ob):
            csels = tuple(cv[h, pl.ds(nl0, LANES)] for nl0 in range(0, NB, LANES))

            @plsc.parallel_loop(0, d, unroll=8)
            def col(c):
                for g in range(NB // LANES):
                    vals = plsc.load_gather(gbufs[b], [nlvecs[g], csels[g] + c])
                    obufs[ob][c, pl.ds(g * LANES, LANES)] = vals

        for b in range(4):
            gfire(b, b)

        def step(h, b, ob):
            @pl.when(h >= 2)
            def _():
                swait(h - 2, ob)

            gwait(h, b)
            compact(h, b, ob)
            sfire(h, ob)

            @pl.when(h + 4 < hist)
            def _():
                gfire(h + 4, b)

        def group(g, carry):
            for b in range(4):
                h = 4 * g + b
                step(h, b, b % 2)
            return carry

        lax.fori_loop(0, hist // 4, group, 0)
        for h in range(hist - hist % 4, hist):
            step(h, h % 4, h % 2)

        swait(hist - 2, (hist - 2) % 2)
        swait(hist - 1, (hist - 1) % 2)

    return body(vidx, csel, wv)


def kernel(input, weight):
    batch, hist = input.shape
    v, d = weight.shape
    assert batch == NUM_WORKERS * NB and hist % 2 == 0
    idxT = input.T
    vidx = idxT >> 1
    csel = (idxT & 1) * d
    wv = _tc_transpose(weight.T, v=v, d=d)
    out3 = _gather_rows(vidx, csel, wv, hist=hist, d=d)
    return jnp.transpose(out3, (2, 0, 1))
